# Initial kernel scaffold; baseline (speedup 1.0000x reference)
import functools

import jax
import jax.numpy as jnp
from jax import lax
from jax.experimental import pallas as pl
from jax.experimental.pallas import tpu as pltpu
from jax.experimental.pallas import tpu_sc as plsc

F32 = jnp.float32
I32 = jnp.int32

N_PEDS = 10000
N_EDGES = 10000
NNZ = 80000
FEAT = 96
ATT_DIM = 32
IN_C = 128
OUT_C = 128
NPAIRS = 8

NC = 2
NS = 16
PAIRS_PER_CORE = NPAIRS // NC
NPAD = 10240
ROWS_PER_TILE = NPAD // NS
NNZ_PER_TILE = NNZ // NS
CHUNK = 125
NCHUNK = NNZ_PER_TILE // CHUNK
RCHUNK = 64
NRCHUNK = ROWS_PER_TILE // RCHUNK
NLANE = 16


def _tc_matmul_body(x_ref, att_ref, wtop_ref, wbot_ref, out_ref):
    xb = x_ref[0, :, 0, :]
    ab = att_ref[0, 0]
    top = lax.dot_general(xb, wtop_ref[...], (((0,), (0,)), ((), ())),
                          preferred_element_type=F32)
    bot = lax.dot_general(ab, wbot_ref[...], (((1,), (0,)), ((), ())),
                          preferred_element_type=F32)
    out_ref[0] = top + bot


def _tc_matmul(x, att, lin_w):
    b, f, t, n = x.shape
    wtop = lin_w[:FEAT]
    wbot = lin_w[FEAT:]
    return pl.pallas_call(
        _tc_matmul_body,
        grid=(NPAIRS,),
        in_specs=[
            pl.BlockSpec((1, FEAT, 1, n), lambda p: (p // 4, 0, p % 4, 0)),
            pl.BlockSpec((1, 1, n, ATT_DIM), lambda p: (p // 4, p % 4, 0, 0)),
            pl.BlockSpec((FEAT, OUT_C), lambda p: (0, 0)),
            pl.BlockSpec((ATT_DIM, OUT_C), lambda p: (0, 0)),
        ],
        out_specs=pl.BlockSpec((1, n, OUT_C), lambda p: (p, 0, 0)),
        out_shape=jax.ShapeDtypeStruct((NPAIRS, n, OUT_C), F32),
    )(x, att, wtop, wbot)


def _sc_body(xl, nig, nil, eil, eig, bias_h,
             out_h, edge_h,
             table, bcnt, dcnt,
             nig_v, nil_v, eil_v, eig_v,
             rowbuf, ones_v, rowchunk, zerochunk, cnt_v, zcnt, bias_v,
             sem):
    c = lax.axis_index("c")
    s = lax.axis_index("s")
    r0 = s * ROWS_PER_TILE

    zeros16 = jnp.zeros((NLANE,), F32)
    ones16 = jnp.ones((NLANE,), F32)

    @pl.loop(0, RCHUNK)
    def _(r):
        for cc in range(OUT_C // NLANE):
            zerochunk[r, pl.ds(cc * NLANE, NLANE)] = zeros16

    @pl.loop(0, ROWS_PER_TILE // NLANE)
    def _(i):
        zcnt[pl.ds(i * NLANE, NLANE)] = zeros16

    @pl.loop(0, 8)
    def _(i):
        ones_v[pl.ds(i * NLANE, NLANE)] = ones16

    pltpu.sync_copy(bias_h, bias_v)
    bias_regs = [bias_v[pl.ds(cc * NLANE, NLANE)] for cc in range(OUT_C // NLANE)]

    @pl.loop(0, NRCHUNK)
    def _(i):
        pltpu.sync_copy(zerochunk, table.at[pl.ds(r0 + i * RCHUNK, RCHUNK)])

    pltpu.sync_copy(zcnt, bcnt.at[pl.ds(r0, ROWS_PER_TILE)])
    pltpu.sync_copy(zcnt, dcnt.at[pl.ds(r0, ROWS_PER_TILE)])
    plsc.subcore_barrier()

    @pl.loop(0, PAIRS_PER_CORE)
    def _(q):
        p = c * PAIRS_PER_CORE + q

        pltpu.sync_copy(nig.at[p, s], nig_v)
        pltpu.sync_copy(nil.at[p, s], nil_v)
        pltpu.sync_copy(eil.at[p, s], eil_v)
        pltpu.sync_copy(eig.at[p, s], eig_v)

        @pl.loop(0, NCHUNK)
        def _(j):
            pltpu.async_copy(xl.at[nig_v.at[j]], rowbuf, sem).wait()
            pltpu.sync_copy(rowbuf, table.at[eil_v.at[j]], add=True)
            pltpu.sync_copy(ones_v.at[pl.ds(0, CHUNK)], bcnt.at[eil_v.at[j]],
                            add=True)
            pltpu.sync_copy(ones_v.at[pl.ds(0, CHUNK)], dcnt.at[nil_v.at[j]],
                            add=True)

        plsc.subcore_barrier()

        pltpu.sync_copy(bcnt.at[pl.ds(r0, ROWS_PER_TILE)], cnt_v)

        @pl.loop(0, ROWS_PER_TILE // NLANE)
        def _(k):
            cv = cnt_v[pl.ds(k * NLANE, NLANE)]
            cnt_v[pl.ds(k * NLANE, NLANE)] = 1.0 / jnp.maximum(cv, 1.0)

        ebase = p * NPAD + r0

        @pl.loop(0, NRCHUNK)
        def _(i):
            pltpu.sync_copy(table.at[pl.ds(r0 + i * RCHUNK, RCHUNK)], rowchunk)

            @pl.loop(0, RCHUNK)
            def _(r):
                inv = plsc.load_gather(
                    cnt_v, [jnp.full((NLANE,), i * RCHUNK + r, I32)])
                for cc in range(OUT_C // NLANE):
                    rowchunk[r, pl.ds(cc * NLANE, NLANE)] = (
                        rowchunk[r, pl.ds(cc * NLANE, NLANE)] * inv)

            pltpu.sync_copy(rowchunk, edge_h.at[pl.ds(ebase + i * RCHUNK, RCHUNK)])
            pltpu.sync_copy(zerochunk, table.at[pl.ds(r0 + i * RCHUNK, RCHUNK)])

        pltpu.sync_copy(zcnt, bcnt.at[pl.ds(r0, ROWS_PER_TILE)])
        plsc.subcore_barrier()

        @pl.loop(0, NCHUNK)
        def _(j):
            pltpu.async_copy(edge_h.at[eig_v.at[j]], rowbuf, sem).wait()
            pltpu.sync_copy(rowbuf, table.at[nil_v.at[j]], add=True)

        plsc.subcore_barrier()

        pltpu.sync_copy(dcnt.at[pl.ds(r0, ROWS_PER_TILE)], cnt_v)

        @pl.loop(0, ROWS_PER_TILE // NLANE)
        def _(k):
            cv = cnt_v[pl.ds(k * NLANE, NLANE)]
            cnt_v[pl.ds(k * NLANE, NLANE)] = 1.0 / jnp.maximum(cv, 1.0)

        obase = p * NPAD + r0

        @pl.loop(0, NRCHUNK)
        def _(i):
            pltpu.sync_copy(table.at[pl.ds(r0 + i * RCHUNK, RCHUNK)], rowchunk)

            @pl.loop(0, RCHUNK)
            def _(r):
                inv = plsc.load_gather(
                    cnt_v, [jnp.full((NLANE,), i * RCHUNK + r, I32)])
                for cc in range(OUT_C // NLANE):
                    rowchunk[r, pl.ds(cc * NLANE, NLANE)] = (
                        rowchunk[r, pl.ds(cc * NLANE, NLANE)] * inv
                        + bias_regs[cc])

            pltpu.sync_copy(rowchunk, out_h.at[pl.ds(obase + i * RCHUNK, RCHUNK)])
            pltpu.sync_copy(zerochunk, table.at[pl.ds(r0 + i * RCHUNK, RCHUNK)])

        pltpu.sync_copy(zcnt, dcnt.at[pl.ds(r0, ROWS_PER_TILE)])
        plsc.subcore_barrier()


def _sc_hyperconv(xl_flat, nig, nil, eil, eig, bias):
    mesh = plsc.VectorSubcoreMesh(core_axis_name="c", subcore_axis_name="s",
                                  num_cores=NC, num_subcores=NS)
    f = pl.kernel(
        _sc_body,
        out_type=(
            jax.ShapeDtypeStruct((NPAIRS * NPAD, OUT_C), F32),
            jax.ShapeDtypeStruct((NPAIRS * NPAD, OUT_C), F32),
        ),
        mesh=mesh,
        scratch_types=[
            pltpu.VMEM_SHARED((NPAD, OUT_C), F32),
            pltpu.VMEM_SHARED((NPAD,), F32),
            pltpu.VMEM_SHARED((NPAD,), F32),
            pltpu.VMEM((NCHUNK, CHUNK), I32),
            pltpu.VMEM((NCHUNK, CHUNK), I32),
            pltpu.VMEM((NCHUNK, CHUNK), I32),
            pltpu.VMEM((NCHUNK, CHUNK), I32),
            pltpu.VMEM((CHUNK, OUT_C), F32),
            pltpu.VMEM((128,), F32),
            pltpu.VMEM((RCHUNK, OUT_C), F32),
            pltpu.VMEM((RCHUNK, OUT_C), F32),
            pltpu.VMEM((ROWS_PER_TILE,), F32),
            pltpu.VMEM((ROWS_PER_TILE,), F32),
            pltpu.VMEM((OUT_C,), F32),
            pltpu.SemaphoreType.DMA,
        ],
    )
    return f(xl_flat, nig, nil, eil, eig, bias)


@jax.jit
def kernel(x, H, sequential_scene_attention, W, lin_w, bias):
    b, f, t, n = x.shape
    xl = _tc_matmul(x, sequential_scene_attention, lin_w)
    xl_flat = xl.reshape(NPAIRS * n, OUT_C)

    node = H[:, :, 0, :].reshape(NPAIRS, NS, NCHUNK, CHUNK)
    edge = H[:, :, 1, :].reshape(NPAIRS, NS, NCHUNK, CHUNK)
    poff = jnp.arange(NPAIRS, dtype=I32).reshape(NPAIRS, 1, 1, 1)
    nig = node + poff * n
    eig = edge + poff * NPAD

    out_flat, _ = _sc_hyperconv(xl_flat, nig, node, edge, eig, bias)
    out = out_flat.reshape(NPAIRS, NPAD, OUT_C)[:, :n, :]
    return out.reshape(b, OUT_C, t, n)


# speedup vs baseline: 37.3827x; 37.3827x over previous
import functools

import jax
import jax.numpy as jnp
from jax import lax
from jax.experimental import pallas as pl
from jax.experimental.pallas import tpu as pltpu
from jax.experimental.pallas import tpu_sc as plsc

F32 = jnp.float32
I32 = jnp.int32

N_PEDS = 10000
N_EDGES = 10000
NNZ = 80000
FEAT = 96
ATT_DIM = 32
IN_C = 128
OUT_C = 128
NPAIRS = 8

NC = 2
NS = 16
PAIRS_PER_CORE = NPAIRS // NC
NPAD = 10240
ROWS_PER_TILE = NPAD // NS
NNZ_PER_TILE = NNZ // NS
CHUNK = 125
NCHUNK = NNZ_PER_TILE // CHUNK
RCHUNK = 32
NRCHUNK = ROWS_PER_TILE // RCHUNK
NLANE = 16


def _tc_matmul_body(x_ref, att_ref, wtop_ref, wbot_ref, out_ref):
    xb = x_ref[0, 0]
    ab = att_ref[0, 0]
    top = lax.dot_general(xb, wtop_ref[...], (((0,), (0,)), ((), ())),
                          preferred_element_type=F32)
    bot = lax.dot_general(ab, wbot_ref[...], (((1,), (0,)), ((), ())),
                          preferred_element_type=F32)
    out_ref[0] = top + bot


def _tc_matmul(xt, att, lin_w):
    b, t, f, n = xt.shape
    wtop = lin_w[:FEAT]
    wbot = lin_w[FEAT:]
    return pl.pallas_call(
        _tc_matmul_body,
        grid=(NPAIRS,),
        in_specs=[
            pl.BlockSpec((1, 1, FEAT, n),
                         lambda p: (lax.div(p, 4), lax.rem(p, 4), 0, 0)),
            pl.BlockSpec((1, 1, n, ATT_DIM),
                         lambda p: (lax.div(p, 4), lax.rem(p, 4), 0, 0)),
            pl.BlockSpec((FEAT, OUT_C), lambda p: (0, 0)),
            pl.BlockSpec((ATT_DIM, OUT_C), lambda p: (0, 0)),
        ],
        out_specs=pl.BlockSpec((1, n, OUT_C), lambda p: (p, 0, 0)),
        out_shape=jax.ShapeDtypeStruct((NPAIRS, n, OUT_C), F32),
    )(xt, att, wtop, wbot)


def _sc_body(xl, nig, nil, eil, eig, bias_h,
             out_h, edge_h,
             table, bcnt, dcnt,
             nig_v, nil_v, eil_v, eig_v,
             rowbuf, ones_v, rowchunk, zerochunk, cnt_v, zcnt, bias_v,
             sem):
    c = lax.axis_index("c")
    s = lax.axis_index("s")
    r0 = s * ROWS_PER_TILE

    zeros16 = jnp.zeros((NLANE,), F32)
    ones16 = jnp.ones((NLANE,), F32)

    @pl.loop(0, RCHUNK)
    def _(r):
        for cc in range(OUT_C // NLANE):
            zerochunk[r, pl.ds(cc * NLANE, NLANE)] = zeros16

    @pl.loop(0, ROWS_PER_TILE // NLANE)
    def _(i):
        zcnt[pl.ds(i * NLANE, NLANE)] = zeros16

    @pl.loop(0, 8)
    def _(i):
        ones_v[pl.ds(i * NLANE, NLANE)] = ones16

    pltpu.sync_copy(bias_h, bias_v)
    bias_regs = [bias_v[pl.ds(cc * NLANE, NLANE)] for cc in range(OUT_C // NLANE)]

    @pl.loop(0, NRCHUNK)
    def _(i):
        pltpu.sync_copy(zerochunk, table.at[pl.ds(r0 + i * RCHUNK, RCHUNK)])

    pltpu.sync_copy(zcnt, bcnt.at[pl.ds(r0, ROWS_PER_TILE)])
    pltpu.sync_copy(zcnt, dcnt.at[pl.ds(r0, ROWS_PER_TILE)])
    plsc.subcore_barrier()

    @pl.loop(0, PAIRS_PER_CORE)
    def _(q):
        p = c * PAIRS_PER_CORE + q

        pltpu.sync_copy(nig.at[p, s], nig_v)
        pltpu.sync_copy(nil.at[p, s], nil_v)
        pltpu.sync_copy(eil.at[p, s], eil_v)
        pltpu.sync_copy(eig.at[p, s], eig_v)

        @pl.loop(0, NCHUNK)
        def _(j):
            pltpu.async_copy(xl.at[nig_v.at[j]], rowbuf, sem).wait()
            pltpu.sync_copy(rowbuf, table.at[eil_v.at[j]], add=True)
            pltpu.sync_copy(ones_v.at[pl.ds(0, CHUNK)], bcnt.at[eil_v.at[j]],
                            add=True)
            pltpu.sync_copy(ones_v.at[pl.ds(0, CHUNK)], dcnt.at[nil_v.at[j]],
                            add=True)

        plsc.subcore_barrier()

        pltpu.sync_copy(bcnt.at[pl.ds(r0, ROWS_PER_TILE)], cnt_v)

        @pl.loop(0, ROWS_PER_TILE // NLANE)
        def _(k):
            cv = cnt_v[pl.ds(k * NLANE, NLANE)]
            cnt_v[pl.ds(k * NLANE, NLANE)] = 1.0 / jnp.maximum(cv, 1.0)

        ebase = p * NPAD + r0

        @pl.loop(0, NRCHUNK)
        def _(i):
            pltpu.sync_copy(table.at[pl.ds(r0 + i * RCHUNK, RCHUNK)], rowchunk)

            @pl.loop(0, RCHUNK // NLANE)
            def _(g):
                invv = cnt_v[pl.ds(i * RCHUNK + g * NLANE, NLANE)]
                for r in range(NLANE):
                    inv = jnp.full((NLANE,), invv[r], F32)
                    row = g * NLANE + r
                    for cc in range(OUT_C // NLANE):
                        rowchunk[row, pl.ds(cc * NLANE, NLANE)] = (
                            rowchunk[row, pl.ds(cc * NLANE, NLANE)] * inv)

            pltpu.sync_copy(rowchunk, edge_h.at[pl.ds(ebase + i * RCHUNK, RCHUNK)])
            pltpu.sync_copy(zerochunk, table.at[pl.ds(r0 + i * RCHUNK, RCHUNK)])

        pltpu.sync_copy(zcnt, bcnt.at[pl.ds(r0, ROWS_PER_TILE)])
        plsc.subcore_barrier()

        @pl.loop(0, NCHUNK)
        def _(j):
            pltpu.async_copy(edge_h.at[eig_v.at[j]], rowbuf, sem).wait()
            pltpu.sync_copy(rowbuf, table.at[nil_v.at[j]], add=True)

        plsc.subcore_barrier()

        pltpu.sync_copy(dcnt.at[pl.ds(r0, ROWS_PER_TILE)], cnt_v)

        @pl.loop(0, ROWS_PER_TILE // NLANE)
        def _(k):
            cv = cnt_v[pl.ds(k * NLANE, NLANE)]
            cnt_v[pl.ds(k * NLANE, NLANE)] = 1.0 / jnp.maximum(cv, 1.0)

        obase = p * NPAD + r0

        @pl.loop(0, NRCHUNK)
        def _(i):
            pltpu.sync_copy(table.at[pl.ds(r0 + i * RCHUNK, RCHUNK)], rowchunk)

            @pl.loop(0, RCHUNK // NLANE)
            def _(g):
                invv = cnt_v[pl.ds(i * RCHUNK + g * NLANE, NLANE)]
                for r in range(NLANE):
                    inv = jnp.full((NLANE,), invv[r], F32)
                    row = g * NLANE + r
                    for cc in range(OUT_C // NLANE):
                        rowchunk[row, pl.ds(cc * NLANE, NLANE)] = (
                            rowchunk[row, pl.ds(cc * NLANE, NLANE)] * inv
                            + bias_regs[cc])

            pltpu.sync_copy(rowchunk, out_h.at[pl.ds(obase + i * RCHUNK, RCHUNK)])
            pltpu.sync_copy(zerochunk, table.at[pl.ds(r0 + i * RCHUNK, RCHUNK)])

        pltpu.sync_copy(zcnt, dcnt.at[pl.ds(r0, ROWS_PER_TILE)])
        plsc.subcore_barrier()


def _sc_hyperconv(xl_flat, nig, nil, eil, eig, bias):
    mesh = plsc.VectorSubcoreMesh(core_axis_name="c", subcore_axis_name="s",
                                  num_cores=NC, num_subcores=NS)
    f = pl.kernel(
        _sc_body,
        out_type=(
            jax.ShapeDtypeStruct((NPAIRS * NPAD, OUT_C), F32),
            jax.ShapeDtypeStruct((NPAIRS * NPAD, OUT_C), F32),
        ),
        mesh=mesh,
        scratch_types=[
            pltpu.VMEM_SHARED((NPAD, OUT_C), F32),
            pltpu.VMEM_SHARED((NPAD,), F32),
            pltpu.VMEM_SHARED((NPAD,), F32),
            pltpu.VMEM((NCHUNK, CHUNK), I32),
            pltpu.VMEM((NCHUNK, CHUNK), I32),
            pltpu.VMEM((NCHUNK, CHUNK), I32),
            pltpu.VMEM((NCHUNK, CHUNK), I32),
            pltpu.VMEM((CHUNK, OUT_C), F32),
            pltpu.VMEM((128,), F32),
            pltpu.VMEM((RCHUNK, OUT_C), F32),
            pltpu.VMEM((RCHUNK, OUT_C), F32),
            pltpu.VMEM((ROWS_PER_TILE,), F32),
            pltpu.VMEM((ROWS_PER_TILE,), F32),
            pltpu.VMEM((OUT_C,), F32),
            pltpu.SemaphoreType.DMA,
        ],
    )
    return f(xl_flat, nig, nil, eil, eig, bias)


@jax.jit
def kernel(x, H, sequential_scene_attention, W, lin_w, bias):
    b, f, t, n = x.shape
    xt = jnp.transpose(x, (0, 2, 1, 3))
    xl = _tc_matmul(xt, sequential_scene_attention, lin_w)
    xl_flat = xl.reshape(NPAIRS * n, OUT_C)

    node = H[:, :, 0, :].reshape(NPAIRS, NS, NCHUNK, CHUNK)
    edge = H[:, :, 1, :].reshape(NPAIRS, NS, NCHUNK, CHUNK)
    poff = jnp.arange(NPAIRS, dtype=I32).reshape(NPAIRS, 1, 1, 1)
    nig = node + poff * n
    eig = edge + poff * NPAD

    out_flat, _ = _sc_hyperconv(xl_flat, nig, node, edge, eig, bias)
    out = out_flat.reshape(NPAIRS, NPAD, OUT_C)[:, :n, :]
    return out.reshape(b, OUT_C, t, n)
